# baseline (speedup 1.0000x reference)
import jax
import jax.numpy as jnp
from jax import lax
from jax.experimental import pallas as pl
from jax.experimental.pallas import tpu as pltpu
from jax.experimental.pallas import tpu_sc as plsc

D_MODEL = 2048
SCALE = 12.0
N_TOKENS = 4 * 4096

NUM_CORES = 2
NUM_SUBCORES = 16
LANES = 16
NW = NUM_CORES * NUM_SUBCORES
B_PER_W = N_TOKENS // NW
CHUNK = 8
NBUF = 3
NCH = B_PER_W // CHUNK
VECS_PER_ROW = D_MODEL // LANES


def _emb_body(ids_hbm, table_hbm, out_hbm, idx_v,
              gb0, gb1, gb2, sb0, sb1, sb2,
              g0, g1, g2, s0, s1, s2):
    gbufs = (gb0, gb1, gb2)
    sbufs = (sb0, sb1, sb2)
    gsems = (g0, g1, g2)
    ssems = (s0, s1, s2)

    wid = lax.axis_index("s") * NUM_CORES + lax.axis_index("c")
    base = wid * B_PER_W
    pltpu.sync_copy(ids_hbm.at[pl.ds(base, B_PER_W)], idx_v)

    def start_gather(c, b):
        pltpu.async_copy(
            table_hbm.at[idx_v.at[pl.ds(c * CHUNK, CHUNK)]],
            gbufs[b], gsems[b])

    def drain_gather(b):
        pltpu.make_async_copy(
            table_hbm.at[idx_v.at[pl.ds(0, CHUNK)]],
            gbufs[b], gsems[b]).wait()

    def scale(b):
        src = gbufs[b]
        dst = sbufs[b]

        for r in range(CHUNK):
            @pl.loop(0, VECS_PER_ROW // 4)
            def _vec(j):
                for k in range(4):
                    sl = pl.ds(j * 4 * LANES + k * LANES, LANES)
                    dst[r, sl] = src[r, sl] * SCALE

    def start_scatter(c, b):
        pltpu.async_copy(
            sbufs[b], out_hbm.at[pl.ds(base + c * CHUNK, CHUNK)], ssems[b])

    def drain_scatter(b):
        pltpu.make_async_copy(
            sbufs[b], out_hbm.at[pl.ds(base, CHUNK)], ssems[b]).wait()

    def step(c, b, drain_s, next_g):
        drain_gather(b)
        if next_g:
            start_gather(c + 2, (b + 2) % NBUF)
        if drain_s:
            drain_scatter(b)
        scale(b)
        start_scatter(c, b)

    start_gather(0, 0)
    start_gather(1, 1)
    for c in range(3):
        step(c, c % NBUF, False, True)

    @pl.loop(1, 20)
    def _round(g):
        for b in range(3):
            step(g * 3 + b, b, True, True)

    step(60, 0, True, True)
    step(61, 1, True, True)
    step(62, 2, True, False)
    step(63, 0, True, False)
    drain_scatter(1)
    drain_scatter(2)
    drain_scatter(0)


@jax.jit
def _embed(ids_flat, embed_table):
    mesh = plsc.VectorSubcoreMesh(
        core_axis_name="c", subcore_axis_name="s",
        num_cores=NUM_CORES, num_subcores=NUM_SUBCORES,
    )
    run = pl.kernel(
        _emb_body,
        out_type=jax.ShapeDtypeStruct((N_TOKENS, D_MODEL), jnp.float32),
        mesh=mesh,
        scratch_types=(
            [pltpu.VMEM((B_PER_W,), jnp.int32)]
            + [pltpu.VMEM((CHUNK, D_MODEL), jnp.float32)] * (2 * NBUF)
            + [pltpu.SemaphoreType.DMA] * (2 * NBUF)
        ),
    )
    return run(ids_flat, embed_table)


def kernel(input_ids, embed_table):
    b, s = input_ids.shape
    ids_flat = input_ids.reshape(-1).astype(jnp.int32)
    out = _embed(ids_flat, embed_table)
    return out.reshape(b, s, D_MODEL)


# speedup vs baseline: 1.7586x; 1.7586x over previous
import jax
import jax.numpy as jnp
from jax import lax
from jax.experimental import pallas as pl
from jax.experimental.pallas import tpu as pltpu
from jax.experimental.pallas import tpu_sc as plsc

D_MODEL = 2048
SCALE = 12.0
N_TOKENS = 4 * 4096

NUM_CORES = 2
NUM_SUBCORES = 16
LANES = 16
NW = NUM_CORES * NUM_SUBCORES
B_PER_W = N_TOKENS // NW
CHUNK = 8
NGBUF = 4
NSBUF = 2
NCH = B_PER_W // CHUNK
VECS_PER_ROW = D_MODEL // LANES


def _emb_body(ids_hbm, table_hbm, out_hbm, idx_v,
              gb0, gb1, gb2, gb3, sb0, sb1,
              g0, g1, g2, g3, s0, s1):
    gbufs = (gb0, gb1, gb2, gb3)
    sbufs = (sb0, sb1)
    gsems = (g0, g1, g2, g3)
    ssems = (s0, s1)

    wid = lax.axis_index("s") * NUM_CORES + lax.axis_index("c")
    base = wid * B_PER_W
    pltpu.sync_copy(ids_hbm.at[pl.ds(base, B_PER_W)], idx_v)

    def start_gather(c, b):
        pltpu.async_copy(
            table_hbm.at[idx_v.at[pl.ds(c * CHUNK, CHUNK)]],
            gbufs[b], gsems[b])

    def drain_gather(b):
        pltpu.make_async_copy(
            table_hbm.at[idx_v.at[pl.ds(0, CHUNK)]],
            gbufs[b], gsems[b]).wait()

    def scale(gb, sb):
        src = gbufs[gb]
        dst = sbufs[sb]

        for r in range(CHUNK):
            @pl.loop(0, VECS_PER_ROW // 8)
            def _vec(j):
                for k in range(8):
                    sl = pl.ds(j * 8 * LANES + k * LANES, LANES)
                    dst[r, sl] = src[r, sl] * SCALE

    def start_scatter(c, b):
        pltpu.async_copy(
            sbufs[b], out_hbm.at[pl.ds(base + c * CHUNK, CHUNK)], ssems[b])

    def drain_scatter(b):
        pltpu.make_async_copy(
            sbufs[b], out_hbm.at[pl.ds(base, CHUNK)], ssems[b]).wait()

    def step(c, gb, sb, drain_s, next_g):
        drain_gather(gb)
        if next_g:
            start_gather(c + 3, (gb + 3) % NGBUF)
        if drain_s:
            drain_scatter(sb)
        scale(gb, sb)
        start_scatter(c, sb)

    start_gather(0, 0)
    start_gather(1, 1)
    start_gather(2, 2)
    step(0, 0, 0, False, True)
    step(1, 1, 1, False, True)
    step(2, 2, 0, True, True)
    step(3, 3, 1, True, True)

    @pl.loop(1, 15)
    def _round(g):
        for b in range(4):
            step(g * 4 + b, b, b % 2, True, True)

    step(60, 0, 0, True, True)
    step(61, 1, 1, True, False)
    step(62, 2, 0, True, False)
    step(63, 3, 1, True, False)
    drain_scatter(0)
    drain_scatter(1)


@jax.jit
def _embed(ids_flat, embed_table):
    mesh = plsc.VectorSubcoreMesh(
        core_axis_name="c", subcore_axis_name="s",
        num_cores=NUM_CORES, num_subcores=NUM_SUBCORES,
    )
    run = pl.kernel(
        _emb_body,
        out_type=jax.ShapeDtypeStruct((N_TOKENS, D_MODEL), jnp.float32),
        mesh=mesh,
        scratch_types=(
            [pltpu.VMEM((B_PER_W,), jnp.int32)]
            + [pltpu.VMEM((CHUNK, D_MODEL), jnp.float32)] * (NGBUF + NSBUF)
            + [pltpu.SemaphoreType.DMA] * (NGBUF + NSBUF)
        ),
    )
    return run(ids_flat, embed_table)


def kernel(input_ids, embed_table):
    b, s = input_ids.shape
    ids_flat = input_ids.reshape(-1).astype(jnp.int32)
    out = _embed(ids_flat, embed_table)
    return out.reshape(b, s, D_MODEL)


# speedup vs baseline: 1.7698x; 1.0064x over previous
import jax
import jax.numpy as jnp
from jax import lax
from jax.experimental import pallas as pl
from jax.experimental.pallas import tpu as pltpu
from jax.experimental.pallas import tpu_sc as plsc

D_MODEL = 2048
SCALE = 12.0
N_TOKENS = 4 * 4096

NUM_CORES = 2
NUM_SUBCORES = 16
LANES = 16
NW = NUM_CORES * NUM_SUBCORES
B_PER_W = N_TOKENS // NW
CHUNK = 8
NBUF = 3
NCH = B_PER_W // CHUNK
VECS_PER_ROW = D_MODEL // LANES


def _emb_body(ids_hbm, table_hbm, out_hbm, idx_v,
              gb0, gb1, gb2, sb0, sb1, sb2,
              g0, g1, g2, s0, s1, s2):
    gbufs = (gb0, gb1, gb2)
    sbufs = (sb0, sb1, sb2)
    gsems = (g0, g1, g2)
    ssems = (s0, s1, s2)

    wid = lax.axis_index("s") * NUM_CORES + lax.axis_index("c")
    base = wid * B_PER_W
    pltpu.sync_copy(ids_hbm.at[pl.ds(base, B_PER_W)], idx_v)

    def start_gather(c, b):
        pltpu.async_copy(
            table_hbm.at[idx_v.at[pl.ds(c * CHUNK, CHUNK)]],
            gbufs[b], gsems[b])

    def drain_gather(b):
        pltpu.make_async_copy(
            table_hbm.at[idx_v.at[pl.ds(0, CHUNK)]],
            gbufs[b], gsems[b]).wait()

    def scale(b):
        src = gbufs[b]
        dst = sbufs[b]

        for r in range(CHUNK):
            @pl.loop(0, VECS_PER_ROW // 8)
            def _vec(j):
                for k in range(8):
                    sl = pl.ds(j * 8 * LANES + k * LANES, LANES)
                    dst[r, sl] = src[r, sl] * SCALE

    def start_scatter(c, b):
        pltpu.async_copy(
            sbufs[b], out_hbm.at[pl.ds(base + c * CHUNK, CHUNK)], ssems[b])

    def drain_scatter(b):
        pltpu.make_async_copy(
            sbufs[b], out_hbm.at[pl.ds(base, CHUNK)], ssems[b]).wait()

    def step(c, b, drain_s, next_g):
        drain_gather(b)
        if next_g:
            start_gather(c + 2, (b + 2) % NBUF)
        if drain_s:
            drain_scatter(b)
        scale(b)
        start_scatter(c, b)

    start_gather(0, 0)
    start_gather(1, 1)
    for c in range(3):
        step(c, c % NBUF, False, True)

    @pl.loop(1, 20)
    def _round(g):
        for b in range(3):
            step(g * 3 + b, b, True, True)

    step(60, 0, True, True)
    step(61, 1, True, True)
    step(62, 2, True, False)
    step(63, 0, True, False)
    drain_scatter(1)
    drain_scatter(2)
    drain_scatter(0)


@jax.jit
def _embed(ids_flat, embed_table):
    mesh = plsc.VectorSubcoreMesh(
        core_axis_name="c", subcore_axis_name="s",
        num_cores=NUM_CORES, num_subcores=NUM_SUBCORES,
    )
    run = pl.kernel(
        _emb_body,
        out_type=jax.ShapeDtypeStruct((N_TOKENS, D_MODEL), jnp.float32),
        mesh=mesh,
        scratch_types=(
            [pltpu.VMEM((B_PER_W,), jnp.int32)]
            + [pltpu.VMEM((CHUNK, D_MODEL), jnp.float32)] * (2 * NBUF)
            + [pltpu.SemaphoreType.DMA] * (2 * NBUF)
        ),
    )
    return run(ids_flat, embed_table)


def kernel(input_ids, embed_table):
    b, s = input_ids.shape
    ids_flat = input_ids.reshape(-1).astype(jnp.int32)
    out = _embed(ids_flat, embed_table)
    return out.reshape(b, s, D_MODEL)
